# baseline (speedup 1.0000x reference)
import functools

import jax
import jax.numpy as jnp
from jax import lax
from jax.experimental import pallas as pl
from jax.experimental.pallas import tpu as pltpu
from jax.experimental.pallas import tpu_sc as plsc

D = 32
L = 16
NW = 32
CHUNK = 128


def _rsqrt(n):
    i = plsc.bitcast(n, jnp.int32)
    i = jnp.int32(0x5F3759DF) - (i >> 1)
    y = plsc.bitcast(i, jnp.float32)
    for _ in range(3):
        y = y * (jnp.float32(1.5) - jnp.float32(0.5) * n * y * y)
    return y


def kernel(x, weight):
    B0, S = x.shape
    N = B0 * S
    per_w = N // NW
    n_chunks = per_w // CHUNK
    xf = x.reshape(NW * n_chunks, CHUNK)
    wp = jnp.pad(weight, ((0, 0), (0, 96)))

    mesh = plsc.VectorSubcoreMesh(core_axis_name="c", subcore_axis_name="s")

    NB = 4
    n_rounds = n_chunks // NB

    @functools.partial(
        pl.kernel,
        mesh=mesh,
        compiler_params=pltpu.CompilerParams(
            needs_layout_passes=False, use_tc_tiling_on_sc=False
        ),
        out_type=jax.ShapeDtypeStruct((N, D), jnp.float32),
        scratch_types=(
            [pltpu.VMEM((n_chunks, CHUNK), jnp.int32)]
            + [pltpu.VMEM((CHUNK, 128), jnp.float32) for _ in range(NB)]
            + [pltpu.SemaphoreType.DMA for _ in range(2 * NB)]
        ),
    )
    def run(x_hbm, tab_hbm, out_hbm, idx_v, *scratch):
        bufs = scratch[:NB]
        sin = scratch[NB : 2 * NB]
        sout = scratch[2 * NB :]
        wid = lax.axis_index("s") * 2 + lax.axis_index("c")
        pltpu.sync_copy(x_hbm.at[pl.ds(wid * n_chunks, n_chunks)], idx_v)
        lanes = lax.iota(jnp.int32, 16)

        def normalize(buf):
            for g in range(CHUNK // L):
                rows = lanes + jnp.int32(g * L)
                vs = []
                acc = jnp.zeros((L,), jnp.float32)
                for d in range(D):
                    col = jnp.full((L,), d, jnp.int32)
                    v = plsc.load_gather(buf, [rows, col])
                    acc = acc + v * v
                    vs.append(v)
                y = _rsqrt(jnp.maximum(acc, jnp.float32(1e-24)))
                for d in range(D):
                    col = jnp.full((L,), d, jnp.int32)
                    plsc.store_scatter(buf, [rows, col], vs[d] * y)

        def out_slice(c):
            return out_hbm.at[pl.ds((wid * n_chunks + c) * CHUNK, CHUNK)]

        def buf_data(b):
            return bufs[b].at[pl.ds(0, CHUNK), pl.ds(0, D)]

        def round_body(r, carry):
            for b in range(NB):
                pltpu.async_copy(tab_hbm.at[idx_v.at[r * NB + b]], bufs[b], sin[b])
            for b in range(NB):
                c = r * NB + b
                pltpu.make_async_copy(
                    tab_hbm.at[idx_v.at[c]], bufs[b], sin[b]
                ).wait()
                normalize(bufs[b])
                pltpu.async_copy(buf_data(b), out_slice(c), sout[b])
            for b in range(NB):
                pltpu.make_async_copy(
                    buf_data(b), out_slice(r * NB + b), sout[b]
                ).wait()
            return carry

        lax.fori_loop(0, n_rounds, round_body, jnp.int32(0))

    out = run(xf, wp)
    return out.reshape(B0, S, D)


# speedup vs baseline: 1.4262x; 1.4262x over previous
import functools

import jax
import jax.numpy as jnp
from jax import lax
from jax.experimental import pallas as pl
from jax.experimental.pallas import tpu as pltpu
from jax.experimental.pallas import tpu_sc as plsc

D = 32
L = 16
NW = 32
CHUNK = 128


def _rsqrt(n):
    i = plsc.bitcast(n, jnp.int32)
    i = jnp.int32(0x5F3759DF) - (i >> 1)
    y = plsc.bitcast(i, jnp.float32)
    for _ in range(3):
        y = y * (jnp.float32(1.5) - jnp.float32(0.5) * n * y * y)
    return y


def kernel(x, weight):
    B0, S = x.shape
    N = B0 * S
    per_w = N // NW
    n_chunks = per_w // CHUNK
    n_tiles = N // CHUNK
    xf = x.reshape(NW * n_chunks, CHUNK)

    NB = 4
    n_rounds = n_chunks // NB

    mesh = plsc.VectorSubcoreMesh(core_axis_name="c", subcore_axis_name="s")

    @functools.partial(
        pl.kernel,
        mesh=mesh,
        compiler_params=pltpu.CompilerParams(
            needs_layout_passes=False, use_tc_tiling_on_sc=False
        ),
        out_type=jax.ShapeDtypeStruct((D // 8, n_tiles, 8, CHUNK), jnp.float32),
        scratch_types=(
            [pltpu.VMEM((n_chunks, CHUNK), jnp.int32)]
            + [pltpu.VMEM((CHUNK, D), jnp.float32) for _ in range(NB)]
            + [pltpu.VMEM((D // 8, 8, CHUNK), jnp.float32) for _ in range(NB)]
            + [pltpu.SemaphoreType.DMA for _ in range(2 * NB)]
        ),
    )
    def run(x_hbm, tab_hbm, out_hbm, idx_v, *scratch):
        gbufs = scratch[:NB]
        obufs = scratch[NB : 2 * NB]
        sin = scratch[2 * NB : 3 * NB]
        sout = scratch[3 * NB :]
        wid = lax.axis_index("s") * 2 + lax.axis_index("c")
        pltpu.sync_copy(x_hbm.at[pl.ds(wid * n_chunks, n_chunks)], idx_v)
        lanes = lax.iota(jnp.int32, 16)

        def normalize(gbuf, obuf):
            for g in range(CHUNK // L):
                rows = lanes + jnp.int32(g * L)
                vs = []
                acc = jnp.zeros((L,), jnp.float32)
                for d in range(D):
                    col = jnp.full((L,), d, jnp.int32)
                    v = plsc.load_gather(gbuf, [rows, col])
                    acc = acc + v * v
                    vs.append(v)
                y = _rsqrt(jnp.maximum(acc, jnp.float32(1e-24)))
                for d in range(D):
                    plsc.store_scatter(
                        obuf,
                        [
                            jnp.full((L,), d // 8, jnp.int32),
                            jnp.full((L,), d % 8, jnp.int32),
                            rows,
                        ],
                        vs[d] * y,
                    )

        def round_body(r, carry):
            for b in range(NB):
                pltpu.async_copy(
                    tab_hbm.at[idx_v.at[r * NB + b]], gbufs[b], sin[b]
                )
            for b in range(NB):
                c = r * NB + b
                pltpu.make_async_copy(
                    tab_hbm.at[idx_v.at[c]], gbufs[b], sin[b]
                ).wait()
                normalize(gbufs[b], obufs[b])
                pltpu.async_copy(
                    obufs[b], out_hbm.at[:, wid * n_chunks + c], sout[b]
                )
            for b in range(NB):
                pltpu.make_async_copy(
                    obufs[b], out_hbm.at[:, wid * n_chunks + r * NB + b], sout[b]
                ).wait()
            return carry

        lax.fori_loop(0, n_rounds, round_body, jnp.int32(0))

    a = run(xf, weight)
    out = a.transpose(0, 2, 1, 3).reshape(D, N).T.reshape(B0, S, D)
    return out


# speedup vs baseline: 1.6132x; 1.1311x over previous
import functools

import jax
import jax.numpy as jnp
from jax import lax
from jax.experimental import pallas as pl
from jax.experimental.pallas import tpu as pltpu
from jax.experimental.pallas import tpu_sc as plsc

D = 32
L = 16
NW = 32
CHUNK = 128


def _rsqrt(n):
    i = plsc.bitcast(n, jnp.int32)
    i = jnp.int32(0x5F3759DF) - (i >> 1)
    y = plsc.bitcast(i, jnp.float32)
    for _ in range(3):
        y = y * (jnp.float32(1.5) - jnp.float32(0.5) * n * y * y)
    return y


def kernel(x, weight):
    B0, S = x.shape
    N = B0 * S
    per_w = N // NW
    n_chunks = per_w // CHUNK
    n_tiles = N // CHUNK
    xf = x.reshape(NW * n_chunks, CHUNK)

    NB = 8
    n_rounds = n_chunks // NB

    mesh = plsc.VectorSubcoreMesh(core_axis_name="c", subcore_axis_name="s")

    @functools.partial(
        pl.kernel,
        mesh=mesh,
        compiler_params=pltpu.CompilerParams(
            needs_layout_passes=False, use_tc_tiling_on_sc=False
        ),
        out_type=jax.ShapeDtypeStruct((D // 8, n_tiles, 8, CHUNK), jnp.float32),
        scratch_types=(
            [pltpu.VMEM((n_chunks, CHUNK), jnp.int32)]
            + [pltpu.VMEM((CHUNK, D), jnp.float32) for _ in range(NB)]
            + [pltpu.VMEM((D // 8, 8, CHUNK), jnp.float32) for _ in range(NB)]
            + [pltpu.SemaphoreType.DMA for _ in range(2 * NB)]
        ),
    )
    def run(x_hbm, tab_hbm, out_hbm, idx_v, *scratch):
        gbufs = scratch[:NB]
        obufs = scratch[NB : 2 * NB]
        sin = scratch[2 * NB : 3 * NB]
        sout = scratch[3 * NB :]
        wid = lax.axis_index("s") * 2 + lax.axis_index("c")
        pltpu.sync_copy(x_hbm.at[pl.ds(wid * n_chunks, n_chunks)], idx_v)
        lanes = lax.iota(jnp.int32, 16)

        def normalize(gbuf, obuf):
            def group(g, carry):
                rows = lanes + g * L
                vs = []
                acc = jnp.zeros((L,), jnp.float32)
                for d in range(D):
                    col = jnp.full((L,), d, jnp.int32)
                    v = plsc.load_gather(gbuf, [rows, col])
                    acc = acc + v * v
                    vs.append(v)
                y = _rsqrt(jnp.maximum(acc, jnp.float32(1e-24)))
                for d in range(D):
                    plsc.store_scatter(
                        obuf,
                        [
                            jnp.full((L,), d // 8, jnp.int32),
                            jnp.full((L,), d % 8, jnp.int32),
                            rows,
                        ],
                        vs[d] * y,
                    )
                return carry

            lax.fori_loop(0, CHUNK // L, group, jnp.int32(0))

        def round_body(r, carry):
            for b in range(NB):
                pltpu.async_copy(
                    tab_hbm.at[idx_v.at[r * NB + b]], gbufs[b], sin[b]
                )
            for b in range(NB):
                c = r * NB + b
                pltpu.make_async_copy(
                    tab_hbm.at[idx_v.at[c]], gbufs[b], sin[b]
                ).wait()
                normalize(gbufs[b], obufs[b])
                pltpu.async_copy(
                    obufs[b], out_hbm.at[:, wid * n_chunks + c], sout[b]
                )
            for b in range(NB):
                pltpu.make_async_copy(
                    obufs[b], out_hbm.at[:, wid * n_chunks + r * NB + b], sout[b]
                ).wait()
            return carry

        lax.fori_loop(0, n_rounds, round_body, jnp.int32(0))

    a = run(xf, weight)
    out = a.transpose(0, 2, 1, 3).reshape(D, N).T.reshape(B0, S, D)
    return out


# speedup vs baseline: 2.1618x; 1.3401x over previous
import functools

import jax
import jax.numpy as jnp
from jax import lax
from jax.experimental import pallas as pl
from jax.experimental.pallas import tpu as pltpu
from jax.experimental.pallas import tpu_sc as plsc

D = 32
L = 16
NW = 32
CHUNK = 128


def _rsqrt(n):
    i = plsc.bitcast(n, jnp.int32)
    i = jnp.int32(0x5F3759DF) - (i >> 1)
    y = plsc.bitcast(i, jnp.float32)
    for _ in range(3):
        y = y * (jnp.float32(1.5) - jnp.float32(0.5) * n * y * y)
    return y


def _relayout_table(weight):
    V, Dd = weight.shape
    NCOL = V // CHUNK
    TAIL = V - NCOL * CHUNK
    wt = weight.T
    wtail = weight[NCOL * CHUNK :, :].reshape(TAIL * Dd)

    mesh = plsc.VectorSubcoreMesh(core_axis_name="c", subcore_axis_name="s")
    NB2 = 2
    per_w = (NCOL + NW - 1) // NW

    @functools.partial(
        pl.kernel,
        mesh=mesh,
        compiler_params=pltpu.CompilerParams(
            needs_layout_passes=False, use_tc_tiling_on_sc=True
        ),
        out_type=jax.ShapeDtypeStruct((V * Dd,), jnp.float32),
        scratch_types=(
            [pltpu.VMEM((Dd, CHUNK), jnp.float32) for _ in range(NB2)]
            + [pltpu.VMEM((CHUNK * Dd,), jnp.float32) for _ in range(NB2)]
            + [pltpu.VMEM((TAIL * Dd,), jnp.float32)]
            + [pltpu.SemaphoreType.DMA for _ in range(2 * NB2)]
        ),
    )
    def run(wt_hbm, wtail_hbm, out_hbm, *scratch):
        vbufs = scratch[:NB2]
        tbufs = scratch[NB2 : 2 * NB2]
        vtail = scratch[2 * NB2]
        sin = scratch[2 * NB2 + 1 : 2 * NB2 + 1 + NB2]
        sout = scratch[2 * NB2 + 1 + NB2 :]
        wid = lax.axis_index("s") * 2 + lax.axis_index("c")
        lanes = lax.iota(jnp.int32, 16)

        @pl.when(wid == 0)
        def _tail():
            pltpu.sync_copy(wtail_hbm, vtail)
            pltpu.sync_copy(vtail, out_hbm.at[pl.ds(NCOL * CHUNK * Dd, TAIL * Dd)])

        def transpose(vbuf, tbuf):
            def body(s, carry):
                base = s * L * Dd
                for d in range(Dd):
                    v = vbuf[d, pl.ds(s * L, L)]
                    plsc.store_scatter(tbuf, [lanes * Dd + (base + d)], v)
                return carry

            lax.fori_loop(0, CHUNK // L, body, jnp.int32(0))

        def col_of(t, b):
            return wid + (t * NB2 + b) * NW

        def step(t, carry):
            for b in range(NB2):
                @pl.when(col_of(t, b) < NCOL)
                def _():
                    pltpu.async_copy(
                        wt_hbm.at[:, pl.ds(col_of(t, b) * CHUNK, CHUNK)],
                        vbufs[b],
                        sin[b],
                    )
            for b in range(NB2):
                @pl.when(col_of(t, b) < NCOL)
                def _():
                    col = col_of(t, b)
                    pltpu.make_async_copy(
                        wt_hbm.at[:, pl.ds(col * CHUNK, CHUNK)], vbufs[b], sin[b]
                    ).wait()
                    transpose(vbufs[b], tbufs[b])
                    pltpu.async_copy(
                        tbufs[b],
                        out_hbm.at[pl.ds(col * CHUNK * Dd, CHUNK * Dd)],
                        sout[b],
                    )
            for b in range(NB2):
                @pl.when(col_of(t, b) < NCOL)
                def _():
                    col = col_of(t, b)
                    pltpu.make_async_copy(
                        tbufs[b],
                        out_hbm.at[pl.ds(col * CHUNK * Dd, CHUNK * Dd)],
                        sout[b],
                    ).wait()
            return carry

        lax.fori_loop(0, (per_w + NB2 - 1) // NB2, step, jnp.int32(0))

    return run(wt, wtail).reshape(V, Dd)


def kernel(x, weight):
    B0, S = x.shape
    IB = B0 // CHUNK
    PB = IB // NW
    n_chunks = S * PB

    xt = x.T.reshape(S, IB, CHUNK)

    NB = 8
    n_rounds = n_chunks // NB

    mesh = plsc.VectorSubcoreMesh(core_axis_name="c", subcore_axis_name="s")

    @functools.partial(
        pl.kernel,
        mesh=mesh,
        compiler_params=pltpu.CompilerParams(
            needs_layout_passes=False, use_tc_tiling_on_sc=False
        ),
        out_type=jax.ShapeDtypeStruct((S, D // 8, IB, 8, CHUNK), jnp.float32),
        scratch_types=(
            [pltpu.VMEM((S, PB, CHUNK), jnp.int32)]
            + [pltpu.VMEM((CHUNK, D), jnp.float32) for _ in range(NB)]
            + [pltpu.VMEM((D // 8, 8, CHUNK), jnp.float32) for _ in range(NB)]
            + [pltpu.SemaphoreType.DMA for _ in range(2 * NB)]
        ),
    )
    def run(x_hbm, tab_hbm, out_hbm, idx_t, *scratch):
        gbufs = scratch[:NB]
        obufs = scratch[NB : 2 * NB]
        sin = scratch[2 * NB : 3 * NB]
        sout = scratch[3 * NB :]
        wid = lax.axis_index("s") * 2 + lax.axis_index("c")
        pltpu.sync_copy(x_hbm.at[:, pl.ds(wid * PB, PB)], idx_t)
        lanes = lax.iota(jnp.int32, 16)

        def normalize(gbuf, obuf):
            def group(g, carry):
                rows = lanes + g * L
                vs = []
                acc = jnp.zeros((L,), jnp.float32)
                for d in range(D):
                    col = jnp.full((L,), d, jnp.int32)
                    v = plsc.load_gather(gbuf, [rows, col])
                    acc = acc + v * v
                    vs.append(v)
                y = _rsqrt(jnp.maximum(acc, jnp.float32(1e-24)))
                for d in range(D):
                    plsc.store_scatter(
                        obuf,
                        [
                            jnp.full((L,), d // 8, jnp.int32),
                            jnp.full((L,), d % 8, jnp.int32),
                            rows,
                        ],
                        vs[d] * y,
                    )
                return carry

            lax.fori_loop(0, CHUNK // L, group, jnp.int32(0))

        def gather_src(c):
            return tab_hbm.at[idx_t.at[c >> 2, c & 3]]

        def out_slice(c):
            return out_hbm.at[c >> 2, pl.ds(0, D // 8), wid * PB + (c & 3)]

        def round_body(r, carry):
            for b in range(NB):
                pltpu.async_copy(gather_src(r * NB + b), gbufs[b], sin[b])
            for b in range(NB):
                c = r * NB + b
                pltpu.make_async_copy(gather_src(c), gbufs[b], sin[b]).wait()
                normalize(gbufs[b], obufs[b])
                pltpu.async_copy(obufs[b], out_slice(c), sout[b])
            for b in range(NB):
                pltpu.make_async_copy(
                    obufs[b], out_slice(r * NB + b), sout[b]
                ).wait()
            return carry

        lax.fori_loop(0, n_rounds, round_body, jnp.int32(0))

    c5 = run(xt, _relayout_table(weight))
    out = (
        c5.transpose(0, 1, 3, 2, 4)
        .reshape(S, D, B0)
        .transpose(2, 0, 1)
    )
    return out


# speedup vs baseline: 2.2813x; 1.0553x over previous
import functools

import jax
import jax.numpy as jnp
from jax import lax
from jax.experimental import pallas as pl
from jax.experimental.pallas import tpu as pltpu
from jax.experimental.pallas import tpu_sc as plsc

D = 32
L = 16
NW = 32
CHUNK = 128


def _rsqrt(n):
    i = plsc.bitcast(n, jnp.int32)
    i = jnp.int32(0x5F3759DF) - (i >> 1)
    y = plsc.bitcast(i, jnp.float32)
    for _ in range(3):
        y = y * (jnp.float32(1.5) - jnp.float32(0.5) * n * y * y)
    return y


def _relayout_table(weight):
    V, Dd = weight.shape
    NCOL = V // CHUNK
    TAIL = V - NCOL * CHUNK
    wt = weight.T
    wtail = weight[NCOL * CHUNK :, :].reshape(TAIL * Dd)

    mesh = plsc.VectorSubcoreMesh(core_axis_name="c", subcore_axis_name="s")
    NB2 = 6
    per_w = (NCOL + NW - 1) // NW

    @functools.partial(
        pl.kernel,
        mesh=mesh,
        compiler_params=pltpu.CompilerParams(
            needs_layout_passes=False, use_tc_tiling_on_sc=True
        ),
        out_type=jax.ShapeDtypeStruct((V * Dd,), jnp.float32),
        scratch_types=(
            [pltpu.VMEM((Dd, CHUNK), jnp.float32) for _ in range(NB2)]
            + [pltpu.VMEM((CHUNK * Dd,), jnp.float32) for _ in range(NB2)]
            + [pltpu.VMEM((TAIL * Dd,), jnp.float32)]
            + [pltpu.SemaphoreType.DMA for _ in range(2 * NB2)]
        ),
    )
    def run(wt_hbm, wtail_hbm, out_hbm, *scratch):
        vbufs = scratch[:NB2]
        tbufs = scratch[NB2 : 2 * NB2]
        vtail = scratch[2 * NB2]
        sin = scratch[2 * NB2 + 1 : 2 * NB2 + 1 + NB2]
        sout = scratch[2 * NB2 + 1 + NB2 :]
        wid = lax.axis_index("s") * 2 + lax.axis_index("c")
        lanes = lax.iota(jnp.int32, 16)

        @pl.when(wid == 0)
        def _tail():
            pltpu.sync_copy(wtail_hbm, vtail)
            pltpu.sync_copy(vtail, out_hbm.at[pl.ds(NCOL * CHUNK * Dd, TAIL * Dd)])

        def transpose(vbuf, tbuf):
            def body(s, carry):
                base = s * L * Dd
                for d in range(Dd):
                    v = vbuf[d, pl.ds(s * L, L)]
                    plsc.store_scatter(tbuf, [lanes * Dd + (base + d)], v)
                return carry

            lax.fori_loop(0, CHUNK // L, body, jnp.int32(0))

        def col_of(t, b):
            return wid + (t * NB2 + b) * NW

        def step(t, carry):
            for b in range(NB2):
                @pl.when(col_of(t, b) < NCOL)
                def _():
                    pltpu.async_copy(
                        wt_hbm.at[:, pl.ds(col_of(t, b) * CHUNK, CHUNK)],
                        vbufs[b],
                        sin[b],
                    )
            for b in range(NB2):
                @pl.when(col_of(t, b) < NCOL)
                def _():
                    col = col_of(t, b)
                    pltpu.make_async_copy(
                        wt_hbm.at[:, pl.ds(col * CHUNK, CHUNK)], vbufs[b], sin[b]
                    ).wait()
                    transpose(vbufs[b], tbufs[b])
                    pltpu.async_copy(
                        tbufs[b],
                        out_hbm.at[pl.ds(col * CHUNK * Dd, CHUNK * Dd)],
                        sout[b],
                    )
            for b in range(NB2):
                @pl.when(col_of(t, b) < NCOL)
                def _():
                    col = col_of(t, b)
                    pltpu.make_async_copy(
                        tbufs[b],
                        out_hbm.at[pl.ds(col * CHUNK * Dd, CHUNK * Dd)],
                        sout[b],
                    ).wait()
            return carry

        lax.fori_loop(0, (per_w + NB2 - 1) // NB2, step, jnp.int32(0))

    return run(wt, wtail).reshape(V, Dd)


def kernel(x, weight):
    B0, S = x.shape
    IB = B0 // CHUNK
    PB = IB // NW
    n_chunks = S * PB

    xt = x.T.reshape(S, IB, CHUNK)

    NB = 8
    n_rounds = n_chunks // NB

    mesh = plsc.VectorSubcoreMesh(core_axis_name="c", subcore_axis_name="s")

    @functools.partial(
        pl.kernel,
        mesh=mesh,
        compiler_params=pltpu.CompilerParams(
            needs_layout_passes=False, use_tc_tiling_on_sc=False
        ),
        out_type=jax.ShapeDtypeStruct((S, D // 8, IB, 8, CHUNK), jnp.float32),
        scratch_types=(
            [pltpu.VMEM((S, PB, CHUNK), jnp.int32)]
            + [pltpu.VMEM((CHUNK, D), jnp.float32) for _ in range(NB)]
            + [pltpu.VMEM((D // 8, 8, CHUNK), jnp.float32) for _ in range(NB)]
            + [pltpu.SemaphoreType.DMA for _ in range(2 * NB)]
        ),
    )
    def run(x_hbm, tab_hbm, out_hbm, idx_t, *scratch):
        gbufs = scratch[:NB]
        obufs = scratch[NB : 2 * NB]
        sin = scratch[2 * NB : 3 * NB]
        sout = scratch[3 * NB :]
        wid = lax.axis_index("s") * 2 + lax.axis_index("c")
        pltpu.sync_copy(x_hbm.at[:, pl.ds(wid * PB, PB)], idx_t)
        lanes = lax.iota(jnp.int32, 16)

        def normalize(gbuf, obuf):
            def group(g, carry):
                rows = lanes + g * L
                vs = []
                acc = jnp.zeros((L,), jnp.float32)
                for d in range(D):
                    col = jnp.full((L,), d, jnp.int32)
                    v = plsc.load_gather(gbuf, [rows, col])
                    acc = acc + v * v
                    vs.append(v)
                y = _rsqrt(jnp.maximum(acc, jnp.float32(1e-24)))
                for d in range(D):
                    plsc.store_scatter(
                        obuf,
                        [
                            jnp.full((L,), d // 8, jnp.int32),
                            jnp.full((L,), d % 8, jnp.int32),
                            rows,
                        ],
                        vs[d] * y,
                    )
                return carry

            lax.fori_loop(0, CHUNK // L, group, jnp.int32(0))

        def gather_src(c):
            return tab_hbm.at[idx_t.at[c >> 2, c & 3]]

        def out_slice(c):
            return out_hbm.at[c >> 2, pl.ds(0, D // 8), wid * PB + (c & 3)]

        def round_body(r, carry):
            for b in range(NB):
                pltpu.async_copy(gather_src(r * NB + b), gbufs[b], sin[b])
            for b in range(NB):
                c = r * NB + b
                pltpu.make_async_copy(gather_src(c), gbufs[b], sin[b]).wait()
                normalize(gbufs[b], obufs[b])
                pltpu.async_copy(obufs[b], out_slice(c), sout[b])
            for b in range(NB):
                pltpu.make_async_copy(
                    obufs[b], out_slice(r * NB + b), sout[b]
                ).wait()
            return carry

        lax.fori_loop(0, n_rounds, round_body, jnp.int32(0))

    c5 = run(xt, _relayout_table(weight))
    out = (
        c5.transpose(0, 1, 3, 2, 4)
        .reshape(S, D, B0)
        .transpose(2, 0, 1)
    )
    return out


# speedup vs baseline: 2.7444x; 1.2030x over previous
import functools

import jax
import jax.numpy as jnp
from jax import lax
from jax.experimental import pallas as pl
from jax.experimental.pallas import tpu as pltpu
from jax.experimental.pallas import tpu_sc as plsc

D = 32
L = 16
NW = 32
CHUNK = 128


def _rsqrt(n):
    i = plsc.bitcast(n, jnp.int32)
    i = jnp.int32(0x5F3759DF) - (i >> 1)
    y = plsc.bitcast(i, jnp.float32)
    for _ in range(3):
        y = y * (jnp.float32(1.5) - jnp.float32(0.5) * n * y * y)
    return y


def kernel(x, weight):
    B0, S = x.shape
    IB = B0 // CHUNK
    PB = IB // NW
    n_chunks = S * PB

    xt = x.T.reshape(S, IB, CHUNK)

    NB = 8
    n_rounds = n_chunks // NB

    mesh = plsc.VectorSubcoreMesh(core_axis_name="c", subcore_axis_name="s")

    @functools.partial(
        pl.kernel,
        mesh=mesh,
        compiler_params=pltpu.CompilerParams(
            needs_layout_passes=False, use_tc_tiling_on_sc=False
        ),
        out_type=jax.ShapeDtypeStruct((S, D // 8, IB, 8, CHUNK), jnp.float32),
        scratch_types=(
            [pltpu.VMEM((S, PB, CHUNK), jnp.int32)]
            + [pltpu.VMEM((CHUNK, D), jnp.float32) for _ in range(NB)]
            + [pltpu.VMEM((D // 8, 8, CHUNK), jnp.float32) for _ in range(NB)]
            + [pltpu.SemaphoreType.DMA for _ in range(2 * NB)]
        ),
    )
    def run(x_hbm, tab_hbm, out_hbm, idx_t, *scratch):
        gbufs = scratch[:NB]
        obufs = scratch[NB : 2 * NB]
        sin = scratch[2 * NB : 3 * NB]
        sout = scratch[3 * NB :]
        wid = lax.axis_index("s") * 2 + lax.axis_index("c")
        pltpu.sync_copy(x_hbm.at[:, pl.ds(wid * PB, PB)], idx_t)
        lanes = lax.iota(jnp.int32, 16)

        def normalize(gbuf, obuf):
            def group(g, carry):
                rows = lanes + g * L
                vs = []
                acc = jnp.zeros((L,), jnp.float32)
                for d in range(D):
                    col = jnp.full((L,), d, jnp.int32)
                    v = plsc.load_gather(gbuf, [rows, col])
                    acc = acc + v * v
                    vs.append(v)
                y = _rsqrt(jnp.maximum(acc, jnp.float32(1e-24)))
                for d in range(D):
                    plsc.store_scatter(
                        obuf,
                        [
                            jnp.full((L,), d // 8, jnp.int32),
                            jnp.full((L,), d % 8, jnp.int32),
                            rows,
                        ],
                        vs[d] * y,
                    )
                return carry

            lax.fori_loop(0, CHUNK // L, group, jnp.int32(0))

        def gather_src(c):
            return tab_hbm.at[idx_t.at[c >> 2, c & 3]]

        def out_slice(c):
            return out_hbm.at[c >> 2, pl.ds(0, D // 8), wid * PB + (c & 3)]

        def round_body(r, carry):
            for b in range(NB):
                pltpu.async_copy(gather_src(r * NB + b), gbufs[b], sin[b])
            for b in range(NB):
                c = r * NB + b
                pltpu.make_async_copy(gather_src(c), gbufs[b], sin[b]).wait()
                normalize(gbufs[b], obufs[b])
                pltpu.async_copy(obufs[b], out_slice(c), sout[b])
            for b in range(NB):
                pltpu.make_async_copy(
                    obufs[b], out_slice(r * NB + b), sout[b]
                ).wait()
            return carry

        lax.fori_loop(0, n_rounds, round_body, jnp.int32(0))

    c5 = run(xt, weight)
    out = (
        c5.transpose(0, 1, 3, 2, 4)
        .reshape(S, D, B0)
        .transpose(2, 0, 1)
    )
    return out
